# Initial kernel scaffold; baseline (speedup 1.0000x reference)
import jax
import jax.numpy as jnp
from jax.experimental import pallas as pl


def kernel(x_nchw, w1, s1, b1, wa, sa, ba, wb, sb, bb, wu, bu):
    raise NotImplementedError("write your pallas kernel here")


# speedup vs baseline: 1.1446x; 1.1446x over previous
import functools

import jax
import jax.numpy as jnp
from jax.experimental import pallas as pl
from jax.experimental.pallas import tpu as pltpu


_TAPS = tuple((oy, ox) for oy in (-1, 0, 1) for ox in (-1, 0, 1))


def _fused_kernel(x_ref, w1_ref, s1_ref, b1_ref,
                  wa_ref, sa_ref, ba_ref,
                  wb_ref, sb_ref, bb_ref,
                  wu_ref, bu_ref, o_ref, *, B, H, W):
    HW = H * W
    pix = jax.lax.broadcasted_iota(jnp.int32, (1, HW), 1)
    py = pix // W
    px = pix % W
    masks = []
    for oy, ox in _TAPS:
        if oy == 0 and ox == 0:
            masks.append(None)
        else:
            masks.append((py + oy >= 0) & (py + oy < H)
                         & (px + ox >= 0) & (px + ox < W))

    def shifted_parts(a_bf):
        ai = pltpu.bitcast(a_bf, jnp.int32)
        parts = []
        for (oy, ox), m in zip(_TAPS, masks):
            if m is None:
                parts.append(a_bf)
                continue
            d = oy * W + ox
            rolled = pltpu.roll(ai, (-d) % HW, axis=1)
            rolled = jnp.where(m, rolled, 0)
            parts.append(pltpu.bitcast(rolled, jnp.bfloat16))
        return parts

    def conv_bn_relu(a_bf, C, w_ref, s_ref, b_ref):
        parts = shifted_parts(a_bf)
        w = w_ref[...]
        scale = s_ref[...]
        bias = b_ref[...]
        outs = []
        for b in range(B):
            pb = jnp.concatenate([p[b * C:(b + 1) * C] for p in parts], axis=0)
            acc = jnp.dot(w, pb, preferred_element_type=jnp.float32)
            outs.append(jnp.maximum(acc * scale + bias, 0.0))
        return outs

    cin = x_ref.shape[1]
    cout = w1_ref.shape[0]
    x_bf = x_ref[...].reshape(B * cin, HW).astype(jnp.bfloat16)
    x1 = conv_bn_relu(x_bf, cin, w1_ref, s1_ref, b1_ref)
    x1_bf = jnp.concatenate([v.astype(jnp.bfloat16) for v in x1], axis=0)
    xa = conv_bn_relu(x1_bf, cout, wa_ref, sa_ref, ba_ref)
    xa_bf = jnp.concatenate([v.astype(jnp.bfloat16) for v in xa], axis=0)
    xb = conv_bn_relu(xa_bf, cout, wb_ref, sb_ref, bb_ref)
    wu = wu_ref[...]
    bu = bu_ref[...]
    for b in range(B):
        s_bf = (xb[b] + x1[b]).astype(jnp.bfloat16)
        y = jnp.dot(wu, s_bf, preferred_element_type=jnp.float32) + bu
        o_ref[b] = y


def _const_spec(shape):
    return pl.BlockSpec(shape, lambda n: (0,) * len(shape))


def kernel(x_nchw, w1, s1, b1, wa, sa, ba, wb, sb, bb, wu, bu):
    N, cin, H, W = x_nchw.shape
    HW = H * W
    cout = w1.shape[0]
    ch4 = wu.shape[0]
    ch = ch4 // 4
    for cand in (8, 6, 4, 3, 2, 1):
        if N % cand == 0:
            B = cand
            break
    bf = jnp.bfloat16

    x3 = x_nchw.reshape(N, cin, HW)
    block_fn = functools.partial(_fused_kernel, B=B, H=H, W=W)

    flops = 2 * N * HW * (9 * cin * cout + 2 * 9 * cout * cout
                          + ch4 * cout)
    bytes_accessed = 4 * (int(x3.size) + N * ch4 * HW) + 2 * (
        w1.size + wa.size + wb.size + wu.size)

    y4 = pl.pallas_call(
        block_fn,
        out_shape=jax.ShapeDtypeStruct((N, ch4, HW), jnp.float32),
        grid=(N // B,),
        in_specs=[
            pl.BlockSpec((B, cin, HW), lambda n: (n, 0, 0)),
            _const_spec((cout, 9 * cin)),
            _const_spec((cout, 1)), _const_spec((cout, 1)),
            _const_spec((cout, 9 * cout)),
            _const_spec((cout, 1)), _const_spec((cout, 1)),
            _const_spec((cout, 9 * cout)),
            _const_spec((cout, 1)), _const_spec((cout, 1)),
            _const_spec((ch4, cout)),
            _const_spec((ch4, 1)),
        ],
        out_specs=pl.BlockSpec((B, ch4, HW), lambda n: (n, 0, 0)),
        compiler_params=pltpu.CompilerParams(
            dimension_semantics=("parallel",),
            vmem_limit_bytes=56 * 1024 * 1024),
        cost_estimate=pl.CostEstimate(flops=flops, transcendentals=0,
                                      bytes_accessed=bytes_accessed),
    )(x3, w1.astype(bf), s1, b1, wa.astype(bf), sa, ba,
      wb.astype(bf), sb, bb, wu.astype(bf), bu)

    y = y4.reshape(N, 2, 2, ch, H, W)
    y = jnp.transpose(y, (0, 3, 4, 1, 5, 2))
    return y.reshape(N, ch, 2 * H, 2 * W)


# speedup vs baseline: 1.1838x; 1.0343x over previous
import functools

import jax
import jax.numpy as jnp
from jax.experimental import pallas as pl
from jax.experimental.pallas import tpu as pltpu


_TAPS = tuple((oy, ox) for oy in (-1, 0, 1) for ox in (-1, 0, 1))


def _fused_kernel(x_ref, w1_ref, s1_ref, b1_ref,
                  wa_ref, sa_ref, ba_ref,
                  wb_ref, sb_ref, bb_ref,
                  wu_ref, bu_ref, o_ref, *, B, H, W):
    HW = H * W
    pix = jax.lax.broadcasted_iota(jnp.int32, (1, HW), 1)
    py = pix // W
    px = pix % W
    masks = []
    for oy, ox in _TAPS:
        if oy == 0 and ox == 0:
            masks.append(None)
        else:
            masks.append((py + oy >= 0) & (py + oy < H)
                         & (px + ox >= 0) & (px + ox < W))

    def shifted_parts(a_bf):
        ai = pltpu.bitcast(a_bf, jnp.int32)
        parts = []
        for (oy, ox), m in zip(_TAPS, masks):
            if m is None:
                parts.append(a_bf)
                continue
            d = oy * W + ox
            rolled = pltpu.roll(ai, (-d) % HW, axis=1)
            rolled = jnp.where(m, rolled, 0)
            parts.append(pltpu.bitcast(rolled, jnp.bfloat16))
        return parts

    def conv_bn_relu(a_bf, C, w_ref, s_ref, b_ref):
        parts = shifted_parts(a_bf)
        w = w_ref[...]
        scale = s_ref[...]
        bias = b_ref[...]
        outs = []
        for b in range(B):
            pb = jnp.concatenate([p[b * C:(b + 1) * C] for p in parts], axis=0)
            acc = jnp.dot(w, pb, preferred_element_type=jnp.float32)
            outs.append(jnp.maximum(acc * scale + bias, 0.0))
        return outs

    cin = x_ref.shape[1]
    cout = w1_ref.shape[0]
    x_bf = x_ref[...].reshape(B * cin, HW).astype(jnp.bfloat16)
    x1 = conv_bn_relu(x_bf, cin, w1_ref, s1_ref, b1_ref)
    x1_bf = jnp.concatenate([v.astype(jnp.bfloat16) for v in x1], axis=0)
    xa = conv_bn_relu(x1_bf, cout, wa_ref, sa_ref, ba_ref)
    xa_bf = jnp.concatenate([v.astype(jnp.bfloat16) for v in xa], axis=0)
    xb = conv_bn_relu(xa_bf, cout, wb_ref, sb_ref, bb_ref)
    wu = wu_ref[...]
    bu = bu_ref[...]
    for b in range(B):
        s_bf = (xb[b] + x1[b]).astype(jnp.bfloat16)
        y = jnp.dot(wu, s_bf, preferred_element_type=jnp.float32) + bu
        o_ref[b] = y.astype(jnp.bfloat16)


def _const_spec(shape):
    return pl.BlockSpec(shape, lambda n: (0,) * len(shape))


def kernel(x_nchw, w1, s1, b1, wa, sa, ba, wb, sb, bb, wu, bu):
    N, cin, H, W = x_nchw.shape
    HW = H * W
    cout = w1.shape[0]
    ch4 = wu.shape[0]
    ch = ch4 // 4
    for cand in (8, 6, 4, 3, 2, 1):
        if N % cand == 0:
            B = cand
            break
    bf = jnp.bfloat16

    x3 = x_nchw.reshape(N, cin, HW)
    block_fn = functools.partial(_fused_kernel, B=B, H=H, W=W)

    flops = 2 * N * HW * (9 * cin * cout + 2 * 9 * cout * cout
                          + ch4 * cout)
    bytes_accessed = 4 * (int(x3.size) + N * ch4 * HW) + 2 * (
        w1.size + wa.size + wb.size + wu.size)

    y4 = pl.pallas_call(
        block_fn,
        out_shape=jax.ShapeDtypeStruct((N, ch4, HW), jnp.bfloat16),
        grid=(N // B,),
        in_specs=[
            pl.BlockSpec((B, cin, HW), lambda n: (n, 0, 0)),
            _const_spec((cout, 9 * cin)),
            _const_spec((cout, 1)), _const_spec((cout, 1)),
            _const_spec((cout, 9 * cout)),
            _const_spec((cout, 1)), _const_spec((cout, 1)),
            _const_spec((cout, 9 * cout)),
            _const_spec((cout, 1)), _const_spec((cout, 1)),
            _const_spec((ch4, cout)),
            _const_spec((ch4, 1)),
        ],
        out_specs=pl.BlockSpec((B, ch4, HW), lambda n: (n, 0, 0)),
        compiler_params=pltpu.CompilerParams(
            dimension_semantics=("parallel",),
            vmem_limit_bytes=56 * 1024 * 1024),
        cost_estimate=pl.CostEstimate(flops=flops, transcendentals=0,
                                      bytes_accessed=bytes_accessed),
    )(x3, w1.astype(bf), s1, b1, wa.astype(bf), sa, ba,
      wb.astype(bf), sb, bb, wu.astype(bf), bu)

    y = y4.reshape(N, 2, 2, ch, H, W)
    y = jnp.transpose(y, (0, 3, 4, 1, 5, 2))
    return y.reshape(N, ch, 2 * H, 2 * W).astype(jnp.float32)
